# Initial kernel scaffold; baseline (speedup 1.0000x reference)
import jax
import jax.numpy as jnp
from jax.experimental import pallas as pl


def kernel(x, table):
    raise NotImplementedError("write your pallas kernel here")


# speedup vs baseline: 1.1710x; 1.1710x over previous
import functools
import math

import jax
import jax.numpy as jnp
from jax import lax
from jax.experimental import pallas as pl
from jax.experimental.pallas import tpu as pltpu
from jax.experimental.pallas import tpu_sc as plsc

_L = 16
_SLAB = 128


@functools.partial(jax.jit, static_argnames=("n", "d"))
def _emb_lookup(idx3, table, n, d):
    info = plsc.get_sparse_core_info()
    nc, ns = info.num_cores, info.num_subcores
    nw = nc * ns
    b_per_w = n // nw
    n_slab = b_per_w // _SLAB
    scale = math.sqrt(float(d))

    mesh = plsc.VectorSubcoreMesh(core_axis_name="c", subcore_axis_name="s")

    @functools.partial(
        pl.kernel,
        mesh=mesh,
        compiler_params=pltpu.CompilerParams(use_tc_tiling_on_sc=False),
        out_type=jax.ShapeDtypeStruct((n, d), jnp.float32),
        scratch_types=[
            pltpu.VMEM((n_slab, _SLAB), jnp.int32),
            pltpu.VMEM((_SLAB, d), jnp.float32),
            pltpu.SemaphoreType.DMA,
        ],
    )
    def k(idx_hbm, table_hbm, out_hbm, idx_v, rows_v, sem):
        wid = lax.axis_index("s") * nc + lax.axis_index("c")
        base = wid * b_per_w
        pltpu.sync_copy(idx_hbm.at[wid], idx_v)

        def slab_body(s, carry):
            pltpu.async_copy(table_hbm.at[idx_v.at[s]], rows_v, sem).wait()

            def mul_body(i, c):
                for jj in range(0, d, _L):
                    rows_v[i, pl.ds(jj, _L)] = rows_v[i, pl.ds(jj, _L)] * scale
                return c

            lax.fori_loop(0, _SLAB, mul_body, 0)
            pltpu.sync_copy(rows_v, out_hbm.at[pl.ds(base + s * _SLAB, _SLAB)])
            return carry

        lax.fori_loop(0, n_slab, slab_body, 0)

    return k(idx3, table)


def kernel(x, table):
    b, h = x.shape
    v, d = table.shape
    n = b * h
    info = plsc.get_sparse_core_info()
    nw = info.num_cores * info.num_subcores
    assert n % (nw * _SLAB) == 0
    b_per_w = n // nw
    idx3 = x.astype(jnp.int32).reshape(nw, b_per_w // _SLAB, _SLAB)
    out = _emb_lookup(idx3, table, n, d)
    return out.reshape(b, h, d)


# speedup vs baseline: 1.4739x; 1.2586x over previous
import functools
import math

import jax
import jax.numpy as jnp
from jax import lax
from jax.experimental import pallas as pl
from jax.experimental.pallas import tpu as pltpu
from jax.experimental.pallas import tpu_sc as plsc

_L = 16
_SLAB = 128
_NBUF = 4


@functools.partial(jax.jit, static_argnames=("n", "d"))
def _emb_lookup(idx3, table, n, d):
    info = plsc.get_sparse_core_info()
    nc, ns = info.num_cores, info.num_subcores
    nw = nc * ns
    b_per_w = n // nw
    n_slab = b_per_w // _SLAB
    n_group = n_slab // _NBUF
    scale = math.sqrt(float(d))

    mesh = plsc.VectorSubcoreMesh(core_axis_name="c", subcore_axis_name="s")

    @functools.partial(
        pl.kernel,
        mesh=mesh,
        compiler_params=pltpu.CompilerParams(use_tc_tiling_on_sc=False),
        out_type=jax.ShapeDtypeStruct((n, d), jnp.float32),
        scratch_types=[
            pltpu.VMEM((n_slab, _SLAB), jnp.int32),
            pltpu.VMEM((_NBUF, _SLAB, d), jnp.float32),
            pltpu.VMEM((_NBUF, _SLAB, d), jnp.float32),
            pltpu.SemaphoreType.DMA((_NBUF,)),
            pltpu.SemaphoreType.DMA((_NBUF,)),
        ],
    )
    def k(idx_hbm, table_hbm, out_hbm, idx_v, gbuf, obuf, gsem, ssem):
        wid = lax.axis_index("s") * nc + lax.axis_index("c")
        base = wid * b_per_w
        pltpu.sync_copy(idx_hbm.at[wid], idx_v)

        for b in range(_NBUF):
            pltpu.async_copy(table_hbm.at[idx_v.at[b]], gbuf.at[b], gsem.at[b])

        def group_body(g, carry):
            for b in range(_NBUF):
                s = g * _NBUF + b
                pltpu.make_async_copy(
                    table_hbm.at[idx_v.at[b]], gbuf.at[b], gsem.at[b]
                ).wait()

                @pl.when(g > 0)
                def _():
                    pltpu.make_async_copy(
                        obuf.at[b], out_hbm.at[pl.ds(0, _SLAB)], ssem.at[b]
                    ).wait()

                def mul_body(i, c):
                    for r in range(4):
                        for jj in range(0, d, _L):
                            obuf[b, i * 4 + r, pl.ds(jj, _L)] = (
                                gbuf[b, i * 4 + r, pl.ds(jj, _L)] * scale
                            )
                    return c

                lax.fori_loop(0, _SLAB // 4, mul_body, 0)

                pltpu.async_copy(
                    obuf.at[b],
                    out_hbm.at[pl.ds(base + s * _SLAB, _SLAB)],
                    ssem.at[b],
                )

                @pl.when(g < n_group - 1)
                def _():
                    pltpu.async_copy(
                        table_hbm.at[idx_v.at[s + _NBUF]], gbuf.at[b], gsem.at[b]
                    )

            return carry

        lax.fori_loop(0, n_group, group_body, 0)

        for b in range(_NBUF):
            pltpu.make_async_copy(
                obuf.at[b], out_hbm.at[pl.ds(0, _SLAB)], ssem.at[b]
            ).wait()

    return k(idx3, table)


def kernel(x, table):
    b, h = x.shape
    v, d = table.shape
    n = b * h
    info = plsc.get_sparse_core_info()
    nw = info.num_cores * info.num_subcores
    assert n % (nw * _SLAB * _NBUF) == 0
    b_per_w = n // nw
    idx3 = x.astype(jnp.int32).reshape(nw, b_per_w // _SLAB, _SLAB)
    out = _emb_lookup(idx3, table, n, d)
    return out.reshape(b, h, d)
